# baseline (speedup 1.0000x reference)
import functools
import math

import jax
import jax.numpy as jnp
from jax import lax
from jax.experimental import pallas as pl
from jax.experimental.pallas import tpu as pltpu
from jax.experimental.pallas import tpu_sc as plsc

B, N, D, H = 16, 256, 256, 8
HD = D // H
S = 8
SH = S * H
KC = 6
PROX = 20.0
SCALE = math.sqrt(float(HD))
SPB = 2
L = 16

_DN = (((1,), (1,)), ((), ()))
_DNS = (((1,), (0,)), ((), ()))


def _dot(a, b, dn=_DNS):
    return lax.dot_general(a, b, dn, preferred_element_type=jnp.float32)


def _sc_topk_body(dist_hbm, rank_hbm, d_v, rk_v, tf_v, ti_v):
    wid = lax.axis_index("s") * 2 + lax.axis_index("c")

    @pl.when(wid < B)
    def _():
        pltpu.sync_copy(dist_hbm.at[wid], d_v)
        iota16 = jnp.arange(L, dtype=jnp.int32)

        def lane_allmin(x, tmp):
            x = jnp.minimum(x, lax.rev(x, (0,)))
            for sh in (4, 2, 1):
                tmp[...] = x
                x = jnp.minimum(
                    x, plsc.load_gather(tmp, [jnp.bitwise_xor(iota16, sh)]))
            return x

        dv = [d_v[pl.ds(c * L, L)] for c in range(N // L)]
        cand_idx = jnp.zeros((L,), jnp.int32)
        for s in range(KC + 1):
            mn = dv[0]
            for c in range(1, N // L):
                mn = jnp.minimum(mn, dv[c])
            mnv = lane_allmin(mn, tf_v)
            cidx = jnp.full((L,), N, jnp.int32)
            for c in range(N // L):
                cidx = jnp.minimum(
                    cidx,
                    jnp.where(dv[c] == mnv, iota16 + c * L, N))
            best = lane_allmin(cidx, ti_v)
            cand_idx = jnp.where(iota16 == s, best, cand_idx)
            for c in range(N // L):
                dv[c] = jnp.where(iota16 + c * L == best, jnp.inf, dv[c])
        seven = jnp.full((L,), KC + 1, jnp.int32)
        for c in range(N // L):
            rk_v[pl.ds(c * L, L)] = seven
        plsc.store_scatter(rk_v, [cand_idx], iota16, mask=iota16 < KC + 1)
        pltpu.sync_copy(rk_v, rank_hbm.at[wid])


def _sc_topk(dist):
    fn = pl.kernel(
        _sc_topk_body,
        mesh=plsc.VectorSubcoreMesh(core_axis_name="c", subcore_axis_name="s"),
        compiler_params=pltpu.CompilerParams(needs_layout_passes=False),
        out_type=jax.ShapeDtypeStruct((B, N), jnp.int32),
        scratch_types=[pltpu.VMEM((N,), jnp.float32),
                       pltpu.VMEM((N,), jnp.int32),
                       pltpu.VMEM((L,), jnp.float32),
                       pltpu.VMEM((L,), jnp.int32)],
    )
    return fn(dist)


QSPB = 4


def _q_kernel(tokens_ref, mask_col_ref, qw2_ref, q_ref):
    tok_flat = tokens_ref[...].reshape(QSPB * N, D)
    q2 = _dot(tok_flat, qw2_ref[...], _DN)
    mask_flat = mask_col_ref[...].reshape(QSPB * N, 1)
    q = jnp.where(mask_flat > 0.0, q2[:, D:], q2[:, :D])
    q_ref[...] = q.reshape(QSPB, N, D)


def _q_project(tokens_B, mask_col, qw2):
    const = lambda b: (0, 0)
    grid_spec = pl.GridSpec(
        grid=(B // QSPB,),
        in_specs=[
            pl.BlockSpec((QSPB, N, D), lambda b: (b, 0, 0)),
            pl.BlockSpec((QSPB, N, 1), lambda b: (b, 0, 0)),
            pl.BlockSpec((2 * D, D), const),
        ],
        out_specs=pl.BlockSpec((QSPB, N, D), lambda b: (b, 0, 0)),
    )
    return pl.pallas_call(
        _q_kernel,
        grid_spec=grid_spec,
        out_shape=jax.ShapeDtypeStruct((B, N, D), jnp.float32),
    )(tokens_B, mask_col, qw2)


def _fused_kernel(dist_full_ref, speed_ref, dist_col_ref,
                  rank_full_ref, rank_col_ref,
                  tokens_ref, q_in_ref, kw_ref, vw_ref, ow_ref,
                  w1t_ref, b1r_ref, w2_ref, b2r_ref, out_ref):
    pid = pl.program_id(0)

    dist_all = dist_full_ref[...]
    close = jnp.sum((dist_all < PROX).astype(jnp.float32))
    avg_density = close / (B * N)
    avg_speed = jnp.mean(speed_ref[...])
    K_t = (4
           + (avg_speed > 15.0).astype(jnp.int32)
           + (avg_density > 0.5).astype(jnp.int32))
    K_t = jnp.minimum(K_t, KC)

    tok_flat = tokens_ref[...].reshape(SPB * N, D)
    q_all = q_in_ref[...].reshape(SPB * N, D)
    dist_flat = dist_col_ref[...].reshape(SPB * N, 1)
    rank_flat = rank_col_ref[...].reshape(SPB * N, 1)

    lane_j_col = lax.broadcasted_iota(jnp.int32, (D, SH), 1)
    d_iota_col = lax.broadcasted_iota(jnp.int32, (D, SH), 0)
    hm = ((d_iota_col // HD) == (lane_j_col % H)).astype(jnp.float32)
    rep = (lax.broadcasted_iota(jnp.int32, (S, SH), 1) // H
           == lax.broadcasted_iota(jnp.int32, (S, SH), 0)
           ).astype(jnp.float32)
    lane_j_row = lax.broadcasted_iota(jnp.int32, (SH, D), 0)
    d_iota_row = lax.broadcasted_iota(jnp.int32, (SH, D), 1)
    hmt = ((d_iota_row // HD) == (lane_j_row % H)).astype(jnp.float32)
    rept = ((lax.broadcasted_iota(jnp.int32, (SH, S), 0) // H)
            == lax.broadcasted_iota(jnp.int32, (SH, S), 1)
            ).astype(jnp.float32)
    g = ((lax.broadcasted_iota(jnp.int32, (SH, SH), 0) % H)
         == (lax.broadcasted_iota(jnp.int32, (SH, SH), 1) % H)
         ).astype(jnp.float32)
    row_s = lax.broadcasted_iota(jnp.int32, (S, N), 0)
    lane_sh = lax.broadcasted_iota(jnp.int32, (N, SH), 1)
    svals = lane_sh // H

    C = D // 4
    T = (KC + 1) * C
    tile64 = (lax.broadcasted_iota(jnp.int32, (C, T), 1) % C
              == lax.broadcasted_iota(jnp.int32, (C, T), 0)
              ).astype(jnp.float32)
    prow = (lax.broadcasted_iota(jnp.int32, (T, C), 0) % C
            == lax.broadcasted_iota(jnp.int32, (T, C), 1)
            ).astype(jnp.float32)
    pcol = (lax.broadcasted_iota(jnp.int32, (H, SH), 1) % H
            == lax.broadcasted_iota(jnp.int32, (H, SH), 0)
            ).astype(jnp.float32)
    blk = (lax.broadcasted_iota(jnp.int32, (T, SH), 0) // C
           == lax.broadcasted_iota(jnp.int32, (T, SH), 1) // H
           ).astype(jnp.float32)
    w2b = _dot(_dot(prow, w2_ref[...], _DN), pcol) * blk
    rep448 = (lax.broadcasted_iota(jnp.int32, (S, T), 1) // C
              == lax.broadcasted_iota(jnp.int32, (S, T), 0)
              ).astype(jnp.float32)
    rb = rep448 * _dot(w1t_ref[1:2, :], tile64)
    b2tile = _dot(b2r_ref[...], pcol)
    a_flat = dist_flat * w1t_ref[0:1, :] + b1r_ref[...]
    a_tiled_flat = _dot(a_flat, tile64)

    lane_ns = lax.broadcasted_iota(jnp.int32, (N, S), 1)

    for sc in range(SPB):
        bg = pid * SPB + sc
        tokens = tok_flat[sc * N:(sc + 1) * N, :]
        q = q_all[sc * N:(sc + 1) * N, :]

        rank_row = rank_full_ref[pl.ds(bg, 1), :]
        r_col = rank_flat[sc * N:(sc + 1) * N, :]
        onehot = ((row_s == rank_row) & (rank_row < KC + 1)
                  ).astype(jnp.float32)
        onehot_t = (lane_ns == r_col).astype(jnp.float32)
        d_row = dist_full_ref[pl.ds(bg, 1), :]

        cand_tok = _dot(onehot, tokens)

        kt = _dot(kw_ref[...], cand_tok, _DN)
        v_cand = _dot(cand_tok, vw_ref[...], _DN)
        khead = _dot(kt, rep) * hm
        vheadt = _dot(rept, v_cand) * hmt

        sf = _dot(q, khead) * (1.0 / SCALE)

        d_cand_row = _dot(d_row, onehot_t)
        b_row = _dot(d_cand_row, rb)
        h1 = jnp.maximum(
            a_tiled_flat[sc * N:(sc + 1) * N, :] + b_row, 0.0)
        bias_all = _dot(h1, w2b) + b2tile

        rank_after_self = svals - (svals > r_col).astype(jnp.int32)
        valid = ((svals != r_col) & (rank_after_self < K_t)
                 & (svals < KC + 1))
        validf = valid.astype(jnp.float32)

        z = jnp.where(valid, sf + bias_all, -1e30)
        m = jnp.max(z, axis=1, keepdims=True)
        e = jnp.exp(z - m) * validf
        denom = _dot(e, g)
        w_all = e / denom

        attn = _dot(w_all, vheadt)
        out_ref[sc] = _dot(attn, ow_ref[...], _DN)


@jax.jit
def kernel(tokens_B, ego_distances, ego_mask, ego_speed, q_w, k_w, v_w,
           ego_q_w, ego_k_w, ego_v_w, out_w, w1, b1, w2, b2):
    del ego_k_w, ego_v_w
    ranks = _sc_topk(ego_distances)
    speed_row = ego_speed.reshape(1, B)
    dist_col = ego_distances.reshape(B, N, 1)
    mask_col = ego_mask.astype(jnp.float32).reshape(B, N, 1)
    rank_col = ranks.reshape(B, N, 1)
    qw2 = jnp.concatenate([q_w, ego_q_w], axis=0)
    q_all = _q_project(tokens_B, mask_col, qw2)
    w1t = w1.T
    b1r = b1.reshape(1, -1)
    b2r = b2.reshape(1, -1)

    const = lambda b: (0, 0)
    grid_spec = pl.GridSpec(
        grid=(B // SPB,),
        in_specs=[
            pl.BlockSpec((B, N), const),
            pl.BlockSpec((1, B), const),
            pl.BlockSpec((SPB, N, 1), lambda b: (b, 0, 0)),
            pl.BlockSpec((B, N), const),
            pl.BlockSpec((SPB, N, 1), lambda b: (b, 0, 0)),
            pl.BlockSpec((SPB, N, D), lambda b: (b, 0, 0)),
            pl.BlockSpec((SPB, N, D), lambda b: (b, 0, 0)),
            pl.BlockSpec((D, D), const),
            pl.BlockSpec((D, D), const),
            pl.BlockSpec((D, D), const),
            pl.BlockSpec((2, D // 4), const),
            pl.BlockSpec((1, D // 4), const),
            pl.BlockSpec((H, D // 4), const),
            pl.BlockSpec((1, H), const),
        ],
        out_specs=pl.BlockSpec((SPB, N, D), lambda b: (b, 0, 0)),
    )
    return pl.pallas_call(
        _fused_kernel,
        grid_spec=grid_spec,
        out_shape=jax.ShapeDtypeStruct((B, N, D), jnp.float32),
    )(ego_distances, speed_row, dist_col, ranks, rank_col,
      tokens_B, q_all, k_w, v_w, out_w, w1t, b1r, w2, b2r)


# speedup vs baseline: 1.0985x; 1.0985x over previous
import functools
import math

import jax
import jax.numpy as jnp
from jax import lax
from jax.experimental import pallas as pl
from jax.experimental.pallas import tpu as pltpu
from jax.experimental.pallas import tpu_sc as plsc

B, N, D, H = 16, 256, 256, 8
HD = D // H
S = 8
SH = S * H
KC = 6
PROX = 20.0
SCALE = math.sqrt(float(HD))
SPB = 2
L = 16

_DN = (((1,), (1,)), ((), ()))
_DNS = (((1,), (0,)), ((), ()))


def _dot(a, b, dn=_DNS):
    return lax.dot_general(a, b, dn, preferred_element_type=jnp.float32)


def _sc_topk_body(dist_hbm, rank_hbm, d_v, rk_v, tf_v, ti_v):
    wid = lax.axis_index("s") + lax.axis_index("c")

    @pl.when(wid < B)
    def _():
        pltpu.sync_copy(dist_hbm.at[wid], d_v)
        iota16 = jnp.arange(L, dtype=jnp.int32)

        def lane_allmin(x, tmp):
            x = jnp.minimum(x, lax.rev(x, (0,)))
            for sh in (4, 2, 1):
                tmp[...] = x
                x = jnp.minimum(
                    x, plsc.load_gather(tmp, [jnp.bitwise_xor(iota16, sh)]))
            return x

        dv = [d_v[pl.ds(c * L, L)] for c in range(N // L)]
        cand_idx = jnp.zeros((L,), jnp.int32)
        for s in range(KC + 1):
            mn = dv[0]
            for c in range(1, N // L):
                mn = jnp.minimum(mn, dv[c])
            mnv = lane_allmin(mn, tf_v)
            cidx = jnp.full((L,), N, jnp.int32)
            for c in range(N // L):
                cidx = jnp.minimum(
                    cidx,
                    jnp.where(dv[c] == mnv, iota16 + c * L, N))
            best = lane_allmin(cidx, ti_v)
            cand_idx = jnp.where(iota16 == s, best, cand_idx)
            for c in range(N // L):
                dv[c] = jnp.where(iota16 + c * L == best, jnp.inf, dv[c])
        seven = jnp.full((L,), KC + 1, jnp.int32)
        for c in range(N // L):
            rk_v[pl.ds(c * L, L)] = seven
        plsc.store_scatter(rk_v, [cand_idx], iota16, mask=iota16 < KC + 1)
        pltpu.sync_copy(rk_v, rank_hbm.at[wid])


def _sc_topk(dist):
    fn = pl.kernel(
        _sc_topk_body,
        mesh=plsc.VectorSubcoreMesh(core_axis_name="c", subcore_axis_name="s",
                                    num_cores=1),
        compiler_params=pltpu.CompilerParams(needs_layout_passes=False),
        out_type=jax.ShapeDtypeStruct((B, N), jnp.int32),
        scratch_types=[pltpu.VMEM((N,), jnp.float32),
                       pltpu.VMEM((N,), jnp.int32),
                       pltpu.VMEM((L,), jnp.float32),
                       pltpu.VMEM((L,), jnp.int32)],
    )
    return fn(dist)


def _fused_kernel(dist_full_ref, speed_ref, dist_col_ref, mask_col_ref,
                  rank_full_ref, rank_col_ref,
                  tokens_ref, qw2_ref, kw_ref, vw_ref, ow_ref,
                  w1t_ref, b1r_ref, w2_ref, b2r_ref, out_ref):
    pid = pl.program_id(0)

    dist_all = dist_full_ref[...]
    close = jnp.sum((dist_all < PROX).astype(jnp.float32))
    avg_density = close / (B * N)
    avg_speed = jnp.mean(speed_ref[...])
    K_t = (4
           + (avg_speed > 15.0).astype(jnp.int32)
           + (avg_density > 0.5).astype(jnp.int32))
    K_t = jnp.minimum(K_t, KC)

    tok_flat = tokens_ref[...].reshape(SPB * N, D)
    q2_all = _dot(tok_flat, qw2_ref[...], _DN)
    mask_flat = mask_col_ref[...].reshape(SPB * N, 1)
    q_all = jnp.where(mask_flat > 0.0, q2_all[:, D:], q2_all[:, :D])
    dist_flat = dist_col_ref[...].reshape(SPB * N, 1)
    rank_flat = rank_col_ref[...].reshape(SPB * N, 1)

    lane_j_col = lax.broadcasted_iota(jnp.int32, (D, SH), 1)
    d_iota_col = lax.broadcasted_iota(jnp.int32, (D, SH), 0)
    hm = ((d_iota_col // HD) == (lane_j_col % H)).astype(jnp.float32)
    rep = (lax.broadcasted_iota(jnp.int32, (S, SH), 1) // H
           == lax.broadcasted_iota(jnp.int32, (S, SH), 0)
           ).astype(jnp.float32)
    lane_j_row = lax.broadcasted_iota(jnp.int32, (SH, D), 0)
    d_iota_row = lax.broadcasted_iota(jnp.int32, (SH, D), 1)
    hmt = ((d_iota_row // HD) == (lane_j_row % H)).astype(jnp.float32)
    rept = ((lax.broadcasted_iota(jnp.int32, (SH, S), 0) // H)
            == lax.broadcasted_iota(jnp.int32, (SH, S), 1)
            ).astype(jnp.float32)
    g = ((lax.broadcasted_iota(jnp.int32, (SH, SH), 0) % H)
         == (lax.broadcasted_iota(jnp.int32, (SH, SH), 1) % H)
         ).astype(jnp.float32)
    row_s = lax.broadcasted_iota(jnp.int32, (S, N), 0)
    lane_sh = lax.broadcasted_iota(jnp.int32, (N, SH), 1)
    svals = lane_sh // H

    C = D // 4
    T = (KC + 1) * C
    tile64 = (lax.broadcasted_iota(jnp.int32, (C, T), 1) % C
              == lax.broadcasted_iota(jnp.int32, (C, T), 0)
              ).astype(jnp.float32)
    prow = (lax.broadcasted_iota(jnp.int32, (T, C), 0) % C
            == lax.broadcasted_iota(jnp.int32, (T, C), 1)
            ).astype(jnp.float32)
    pcol = (lax.broadcasted_iota(jnp.int32, (H, SH), 1) % H
            == lax.broadcasted_iota(jnp.int32, (H, SH), 0)
            ).astype(jnp.float32)
    blk = (lax.broadcasted_iota(jnp.int32, (T, SH), 0) // C
           == lax.broadcasted_iota(jnp.int32, (T, SH), 1) // H
           ).astype(jnp.float32)
    w2b = _dot(_dot(prow, w2_ref[...], _DN), pcol) * blk
    rep448 = (lax.broadcasted_iota(jnp.int32, (S, T), 1) // C
              == lax.broadcasted_iota(jnp.int32, (S, T), 0)
              ).astype(jnp.float32)
    rb = rep448 * _dot(w1t_ref[1:2, :], tile64)
    b2tile = _dot(b2r_ref[...], pcol)
    a_flat = dist_flat * w1t_ref[0:1, :] + b1r_ref[...]
    a_tiled_flat = _dot(a_flat, tile64)

    lane_ns = lax.broadcasted_iota(jnp.int32, (N, S), 1)

    for sc in range(SPB):
        bg = pid * SPB + sc
        tokens = tok_flat[sc * N:(sc + 1) * N, :]
        q = q_all[sc * N:(sc + 1) * N, :]

        rank_row = rank_full_ref[pl.ds(bg, 1), :]
        r_col = rank_flat[sc * N:(sc + 1) * N, :]
        onehot = ((row_s == rank_row) & (rank_row < KC + 1)
                  ).astype(jnp.float32)
        onehot_t = (lane_ns == r_col).astype(jnp.float32)
        d_row = dist_full_ref[pl.ds(bg, 1), :]

        cand_tok = _dot(onehot, tokens)

        kt = _dot(kw_ref[...], cand_tok, _DN)
        v_cand = _dot(cand_tok, vw_ref[...], _DN)
        khead = _dot(kt, rep) * hm
        vheadt = _dot(rept, v_cand) * hmt

        sf = _dot(q, khead) * (1.0 / SCALE)

        d_cand_row = _dot(d_row, onehot_t)
        b_row = _dot(d_cand_row, rb)
        h1 = jnp.maximum(
            a_tiled_flat[sc * N:(sc + 1) * N, :] + b_row, 0.0)
        bias_all = _dot(h1, w2b) + b2tile

        rank_after_self = svals - (svals > r_col).astype(jnp.int32)
        valid = ((svals != r_col) & (rank_after_self < K_t)
                 & (svals < KC + 1))
        validf = valid.astype(jnp.float32)

        z = jnp.where(valid, sf + bias_all, -1e30)
        m = jnp.max(z, axis=1, keepdims=True)
        e = jnp.exp(z - m) * validf
        denom = _dot(e, g)
        w_all = e / denom

        attn = _dot(w_all, vheadt)
        out_ref[sc] = _dot(attn, ow_ref[...], _DN)


@jax.jit
def kernel(tokens_B, ego_distances, ego_mask, ego_speed, q_w, k_w, v_w,
           ego_q_w, ego_k_w, ego_v_w, out_w, w1, b1, w2, b2):
    del ego_k_w, ego_v_w
    ranks = _sc_topk(ego_distances)
    speed_row = ego_speed.reshape(1, B)
    dist_col = ego_distances.reshape(B, N, 1)
    mask_col = ego_mask.astype(jnp.float32).reshape(B, N, 1)
    rank_col = ranks.reshape(B, N, 1)
    qw2 = jnp.concatenate([q_w, ego_q_w], axis=0)
    w1t = w1.T
    b1r = b1.reshape(1, -1)
    b2r = b2.reshape(1, -1)

    const = lambda b: (0, 0)
    grid_spec = pl.GridSpec(
        grid=(B // SPB,),
        in_specs=[
            pl.BlockSpec((B, N), const),
            pl.BlockSpec((1, B), const),
            pl.BlockSpec((SPB, N, 1), lambda b: (b, 0, 0)),
            pl.BlockSpec((SPB, N, 1), lambda b: (b, 0, 0)),
            pl.BlockSpec((B, N), const),
            pl.BlockSpec((SPB, N, 1), lambda b: (b, 0, 0)),
            pl.BlockSpec((SPB, N, D), lambda b: (b, 0, 0)),
            pl.BlockSpec((2 * D, D), const),
            pl.BlockSpec((D, D), const),
            pl.BlockSpec((D, D), const),
            pl.BlockSpec((D, D), const),
            pl.BlockSpec((2, D // 4), const),
            pl.BlockSpec((1, D // 4), const),
            pl.BlockSpec((H, D // 4), const),
            pl.BlockSpec((1, H), const),
        ],
        out_specs=pl.BlockSpec((SPB, N, D), lambda b: (b, 0, 0)),
    )
    return pl.pallas_call(
        _fused_kernel,
        grid_spec=grid_spec,
        out_shape=jax.ShapeDtypeStruct((B, N, D), jnp.float32),
    )(ego_distances, speed_row, dist_col, mask_col, ranks, rank_col,
      tokens_B, qw2, k_w, v_w, out_w, w1t, b1r, w2, b2r)
